# Initial kernel scaffold; baseline (speedup 1.0000x reference)
import jax
import jax.numpy as jnp
from jax.experimental import pallas as pl


def kernel(x, pn_conv1_w, pn_conv1_b, pn_conv2_w, pn_conv2_b, pn_conv3_w, pn_conv3_b, pn_bn1_g, pn_bn1_b, pn_bn2_g, pn_bn2_b, pn_bn3_g, pn_bn3_b, cls_conv1_w, cls_conv1_b, cls_conv2_w, cls_conv2_b, cls_conv3_w, cls_conv3_b, t1_conv1_w, t1_conv1_b, t1_conv2_w, t1_conv2_b, t1_conv3_w, t1_conv3_b, t1_fc1_w, t1_fc1_b, t1_fc2_w, t1_fc2_b, t1_fc3_w, t1_fc3_b, t1_bn1_g, t1_bn1_b, t1_bn2_g, t1_bn2_b, t1_bn3_g, t1_bn3_b, t1_bn4_g, t1_bn4_b, t1_bn5_g, t1_bn5_b, t2_conv1_w, t2_conv1_b, t2_conv2_w, t2_conv2_b, t2_conv3_w, t2_conv3_b, t2_fc1_w, t2_fc1_b, t2_fc2_w, t2_fc2_b, t2_fc3_w, t2_fc3_b, t2_bn1_g, t2_bn1_b, t2_bn2_g, t2_bn2_b, t2_bn3_g, t2_bn3_b, t2_bn4_g, t2_bn4_b, t2_bn5_g, t2_bn5_b):
    raise NotImplementedError("write your pallas kernel here")


# speedup vs baseline: 1.3240x; 1.3240x over previous
import functools

import jax
import jax.numpy as jnp
from jax.experimental import pallas as pl
from jax.experimental.pallas import tpu as pltpu

_EPS = 1e-5


def _bf(a):
    return a.astype(jnp.bfloat16)


def _bff(a):
    return a.astype(jnp.bfloat16).astype(jnp.float32)


def _stats_rows(n, c_out):
    cap = 2048 if c_out <= 256 else 512
    if n <= cap:
        return n
    for t in range(cap, 7, -8):
        if n % t == 0:
            return t
    return n


def _tile_rows(n, chunk, c_out):
    cap = 4096 if c_out <= 256 else 2048
    t = chunk
    while t * 2 <= cap and n % (t * 2) == 0:
        t *= 2
    return t


def _pass_kernel(*refs, stages, want_sq, want_extrema, want_gram,
                 gram_final_h, stats_rows):
    i = 0
    h = refs[i][0]
    i += 1
    hprev, y = h, None
    for st in stages:
        w = refs[i][0] if st["batched"] else refs[i][...]
        i += 1
        hprev = h
        y = jnp.dot(h, w, preferred_element_type=jnp.float32)
        if st["bias"]:
            y = y + refs[i][0]
            i += 1
        hb = y.astype(jnp.bfloat16)
        if st["affine"]:
            sc, sh = refs[i][...], refs[i + 1][...]
            i += 2
            h = _bf(jnp.maximum(hb.astype(jnp.float32) * sc + sh, 0.0))
        else:
            h = hb

    outs = list(refs[i:])
    step = pl.program_id(1)

    @pl.when(step == 0)
    def _():
        k = 0
        if want_sq:
            outs[k][...] = jnp.zeros_like(outs[k])
            outs[k + 1][...] = jnp.zeros_like(outs[k + 1])
            k += 2
        if want_gram:
            outs[k][...] = jnp.zeros_like(outs[k])
            outs[k + 1][...] = jnp.zeros_like(outs[k + 1])
            k += 2
        if want_extrema:
            outs[k][...] = jnp.full(outs[k].shape, -jnp.inf, jnp.float32)
            outs[k + 1][...] = jnp.full(outs[k + 1].shape, jnp.inf, jnp.float32)

    k = 0
    if want_sq:
        ones_row = jnp.ones((1, stats_rows), jnp.float32)
        for j in range(y.shape[0] // stats_rows):
            ys = y[j * stats_rows:(j + 1) * stats_rows]
            outs[k][0] += jnp.dot(ones_row, ys,
                                  preferred_element_type=jnp.float32)
            outs[k + 1][0] += jnp.dot(ones_row, ys * ys,
                                      preferred_element_type=jnp.float32)
        k += 2
    if want_gram:
        hg = h if gram_final_h else hprev
        outs[k][0] += jax.lax.dot_general(
            hg, hg, (((0,), (0,)), ((), ())),
            preferred_element_type=jnp.float32)
        outs[k + 1][0] += jnp.sum(hg.astype(jnp.float32), axis=0,
                                  keepdims=True)
        k += 2
    if want_extrema:
        outs[k][0] = jnp.maximum(outs[k][0], jnp.max(y, axis=0, keepdims=True))
        outs[k + 1][0] = jnp.minimum(outs[k + 1][0],
                                     jnp.min(y, axis=0, keepdims=True))


def _run_pass(x, stage_params, want_sq=False, want_extrema=False,
              want_gram=False, gram_final_h=False):
    b, n, _ = x.shape
    c_last = stage_params[-1][0].shape[-1]
    chunk = _stats_rows(n, c_last)
    tn = _tile_rows(n, chunk, c_last)
    nt = n // tn

    in_specs = [pl.BlockSpec((1, tn, x.shape[2]), lambda bi, ni: (bi, ni, 0))]
    args = [x]
    stages = []
    for w, bias, aff in stage_params:
        c = w.shape[-1]
        if w.ndim == 3:
            in_specs.append(pl.BlockSpec((1,) + w.shape[1:],
                                         lambda bi, ni: (bi, 0, 0)))
        else:
            in_specs.append(pl.BlockSpec(w.shape, lambda bi, ni: (0, 0)))
        args.append(_bf(w))
        if bias is not None:
            in_specs.append(pl.BlockSpec((1, 1, c), lambda bi, ni: (bi, 0, 0)))
            args.append(bias.reshape(b, 1, c).astype(jnp.float32))
        if aff is not None:
            in_specs += [pl.BlockSpec((1, c), lambda bi, ni: (0, 0))] * 2
            args += [aff[0].reshape(1, c).astype(jnp.float32),
                     aff[1].reshape(1, c).astype(jnp.float32)]
        stages.append({"batched": w.ndim == 3, "bias": bias is not None,
                       "affine": aff is not None})

    out_shapes, out_specs = [], []

    def stat_out(c):
        out_shapes.append(jax.ShapeDtypeStruct((b, 1, c), jnp.float32))
        out_specs.append(pl.BlockSpec((1, 1, c), lambda bi, ni: (bi, 0, 0)))

    if want_sq:
        stat_out(c_last)
        stat_out(c_last)
    if want_gram:
        kg = (stage_params[-1][0].shape[-1] if gram_final_h
              else stage_params[-1][0].shape[-2])
        out_shapes.append(jax.ShapeDtypeStruct((b, kg, kg), jnp.float32))
        out_specs.append(pl.BlockSpec((1, kg, kg), lambda bi, ni: (bi, 0, 0)))
        stat_out(kg)
    if want_extrema:
        stat_out(c_last)
        stat_out(c_last)

    fn = functools.partial(_pass_kernel, stages=tuple(stages),
                           want_sq=want_sq, want_extrema=want_extrema,
                           want_gram=want_gram, gram_final_h=gram_final_h,
                           stats_rows=chunk)
    return pl.pallas_call(
        fn, out_shape=tuple(out_shapes), grid=(b, nt),
        in_specs=in_specs, out_specs=tuple(out_specs),
        compiler_params=pltpu.CompilerParams(
            dimension_semantics=("parallel", "arbitrary")),
    )(*args)


def _head_kernel(x_ref, w1_ref, b1_ref, g1_ref, e1_ref, w2_ref, b2_ref,
                 g2_ref, e2_ref, w3_ref, b3_ref, o_ref, *, bn, softmax):
    h = x_ref[...].astype(jnp.float32)
    for w_ref, b_ref, g_ref, e_ref in ((w1_ref, b1_ref, g1_ref, e1_ref),
                                       (w2_ref, b2_ref, g2_ref, e2_ref)):
        y = jnp.dot(_bf(h), w_ref[...],
                    preferred_element_type=jnp.float32) + b_ref[...]
        if bn:
            mu = jnp.mean(y, axis=0, keepdims=True)
            v = jnp.mean(jnp.square(y - mu), axis=0, keepdims=True)
            y = (y - mu) * jax.lax.rsqrt(v + _EPS) * g_ref[...] + e_ref[...]
        h = jnp.maximum(y, 0.0)
    o = jnp.dot(_bf(h), w3_ref[...],
                preferred_element_type=jnp.float32) + b3_ref[...]
    if softmax:
        e = jnp.exp(o - jnp.max(o, axis=1, keepdims=True))
        o = e / jnp.sum(e, axis=1, keepdims=True)
    o_ref[...] = o


def _head(x, w1, b1, g1, e1, w2, b2, g2, e2, w3, b3, *, bn, softmax):
    b = x.shape[0]
    c1, c2, c3 = w1.shape[1], w2.shape[1], w3.shape[1]

    def v(a, c):
        return a.reshape(1, c).astype(jnp.float32)

    args = (x.astype(jnp.float32),
            _bf(w1), v(b1, c1), v(g1, c1), v(e1, c1),
            _bf(w2), v(b2, c2), v(g2, c2), v(e2, c2),
            _bf(w3), v(b3, c3))

    def full(shape):
        nd = len(shape)
        return pl.BlockSpec(shape, lambda i, _nd=nd: (0,) * _nd)

    return pl.pallas_call(
        functools.partial(_head_kernel, bn=bn, softmax=softmax),
        out_shape=jax.ShapeDtypeStruct((b, c3), jnp.float32),
        grid=(1,),
        in_specs=[full(a.shape) for a in args],
        out_specs=full((b, c3)),
        compiler_params=pltpu.CompilerParams(
            dimension_semantics=("arbitrary",)),
    )(*args)


def _stats_to_affine(s, q, cnt, gamma, beta):
    mean = jnp.sum(s, axis=0) / cnt
    var = jnp.maximum(jnp.sum(q, axis=0) / cnt - mean * mean, 0.0)
    sc = gamma * jax.lax.rsqrt(var + _EPS)
    return sc, beta - mean * sc


def _qf(g, w):
    return jnp.einsum("ij,ic,jc->c", g, w, w)


def kernel(x,
           pn_conv1_w, pn_conv1_b, pn_conv2_w, pn_conv2_b, pn_conv3_w, pn_conv3_b,
           pn_bn1_g, pn_bn1_b, pn_bn2_g, pn_bn2_b, pn_bn3_g, pn_bn3_b,
           cls_conv1_w, cls_conv1_b, cls_conv2_w, cls_conv2_b, cls_conv3_w, cls_conv3_b,
           t1_conv1_w, t1_conv1_b, t1_conv2_w, t1_conv2_b, t1_conv3_w, t1_conv3_b,
           t1_fc1_w, t1_fc1_b, t1_fc2_w, t1_fc2_b, t1_fc3_w, t1_fc3_b,
           t1_bn1_g, t1_bn1_b, t1_bn2_g, t1_bn2_b, t1_bn3_g, t1_bn3_b,
           t1_bn4_g, t1_bn4_b, t1_bn5_g, t1_bn5_b,
           t2_conv1_w, t2_conv1_b, t2_conv2_w, t2_conv2_b, t2_conv3_w, t2_conv3_b,
           t2_fc1_w, t2_fc1_b, t2_fc2_w, t2_fc2_b, t2_fc3_w, t2_fc3_b,
           t2_bn1_g, t2_bn1_b, t2_bn2_g, t2_bn2_b, t2_bn3_g, t2_bn3_b,
           t2_bn4_g, t2_bn4_b, t2_bn5_g, t2_bn5_b):
    b, n, pd = x.shape
    cnt = b * n

    xb = jnp.pad(_bf(x), ((0, 0), (0, 0), (0, 8 - pd)))

    w1t = jnp.pad(t1_conv1_w, ((0, 8 - pd), (0, 0)))
    s1, q1 = _run_pass(xb, [(w1t, None, None)], want_sq=True)
    sc1, sh1 = _stats_to_affine(s1[:, 0], q1[:, 0], cnt, t1_bn1_g, t1_bn1_b)

    s2, q2 = _run_pass(xb, [(w1t, None, (sc1, sh1)),
                            (t1_conv2_w, None, None)], want_sq=True)
    sc2, sh2 = _stats_to_affine(s2[:, 0], q2[:, 0], cnt, t1_bn2_g, t1_bn2_b)

    s3, q3, mx1, mn1 = _run_pass(
        xb, [(w1t, None, (sc1, sh1)), (t1_conv2_w, None, (sc2, sh2)),
             (t1_conv3_w, None, None)], want_sq=True, want_extrema=True)
    sc3, sh3 = _stats_to_affine(s3[:, 0], q3[:, 0], cnt, t1_bn3_g, t1_bn3_b)
    ext = jnp.where(sc3 >= 0.0, mx1[:, 0], mn1[:, 0])
    pooled = jnp.maximum(ext * sc3 + sh3, 0.0)

    t1m = (_head(pooled, t1_fc1_w, t1_fc1_b, t1_bn4_g, t1_bn4_b,
                 t1_fc2_w, t1_fc2_b, t1_bn5_g, t1_bn5_b,
                 t1_fc3_w, t1_fc3_b, bn=True, softmax=False)
           + jnp.eye(pd, dtype=jnp.float32).reshape(1, pd * pd)
           ).reshape(b, pd, pd)

    wf1 = jnp.pad(jnp.einsum("bij,jk->bik", t1m, pn_conv1_w),
                  ((0, 0), (0, 8 - pd), (0, 0)))
    s1m, q1m, gy1, csy1 = _run_pass(xb, [(wf1, None, None)],
                                    want_sq=True, want_gram=True,
                                    gram_final_h=True)
    sc1m, sh1m = _stats_to_affine(s1m[:, 0], q1m[:, 0], cnt,
                                  pn_bn1_g, pn_bn1_b)

    w1e = sc1m[:, None] * t2_conv1_w
    sa, qa = _run_pass(xb, [(wf1, None, None), (w1e, None, None)],
                       want_sq=True)
    sca, sha = _stats_to_affine(sa[:, 0], qa[:, 0], cnt, t2_bn1_g, t2_bn1_b)

    sb, qb = _run_pass(xb, [(wf1, None, None), (w1e, None, (sca, sha)),
                            (t2_conv2_w, None, None)], want_sq=True)
    scb, shb = _stats_to_affine(sb[:, 0], qb[:, 0], cnt, t2_bn2_g, t2_bn2_b)

    sc_, qc_, mx2, mn2 = _run_pass(
        xb, [(wf1, None, None), (w1e, None, (sca, sha)),
             (t2_conv2_w, None, (scb, shb)), (t2_conv3_w, None, None)],
        want_sq=True, want_extrema=True)
    scc, shc = _stats_to_affine(sc_[:, 0], qc_[:, 0], cnt, t2_bn3_g, t2_bn3_b)
    ext2 = jnp.where(scc >= 0.0, mx2[:, 0], mn2[:, 0])
    pooled2 = jnp.maximum(ext2 * scc + shc, 0.0)

    t2m = (_head(pooled2, t2_fc1_w, t2_fc1_b, t2_bn4_g, t2_bn4_b,
                 t2_fc2_w, t2_fc2_b, t2_bn5_g, t2_bn5_b,
                 t2_fc3_w, t2_fc3_b, bn=True, softmax=False)
           + jnp.eye(64, dtype=jnp.float32).reshape(1, 64 * 64)
           ).reshape(b, 64, 64)

    t2w2 = jnp.einsum("bij,jc->bic", t2m, pn_conv2_w)
    wf2 = sc1m[None, :, None] * t2w2
    bf2 = jnp.einsum("j,bjc->bc", sh1m, t2w2) + pn_conv2_b
    wf2f = _bff(wf2)
    lin = jnp.einsum("bi,bic->bc", csy1[:, 0], wf2f)
    s2m = jnp.sum(lin + n * bf2, axis=0)
    q2m = jnp.sum(jnp.einsum("bij,bic,bjc->bc", gy1, wf2f, wf2f)
                  + 2.0 * bf2 * lin + n * bf2 * bf2, axis=0)
    mean2 = s2m / cnt
    var2 = jnp.maximum(q2m / cnt - mean2 * mean2, 0.0)
    sc2m = pn_bn2_g * jax.lax.rsqrt(var2 + _EPS)
    wf3 = sc2m[:, None] * pn_conv3_w

    g2m, cs2m, mx3, mn3 = _run_pass(
        xb, [(wf1, None, None), (wf2, bf2, None), (wf3, None, None)],
        want_gram=True, want_extrema=True)
    wf3f = _bff(wf3)
    s3m = jnp.sum(cs2m[:, 0], axis=0) @ wf3f
    q3m = _qf(jnp.sum(g2m, axis=0), wf3f)
    mean3 = s3m / cnt
    var3 = jnp.maximum(q3m / cnt - mean3 * mean3, 0.0)
    sc3m = pn_bn3_g * jax.lax.rsqrt(var3 + _EPS)
    sh3m = pn_bn3_b - mean3 * sc3m
    pooled3 = jnp.where(sc3m >= 0.0, mx3[:, 0], mn3[:, 0]) * sc3m + sh3m

    ones = jnp.ones((cls_conv1_w.shape[1],), jnp.float32)
    zeros = jnp.zeros_like(ones)
    c2w = cls_conv2_w.shape[1]
    out = _head(pooled3, cls_conv1_w, cls_conv1_b, ones, zeros,
                cls_conv2_w, cls_conv2_b, ones[:c2w], zeros[:c2w],
                cls_conv3_w, cls_conv3_b, bn=False, softmax=True)
    return out.reshape(b, cls_conv3_w.shape[1], 1)


# speedup vs baseline: 1.7855x; 1.3486x over previous
import functools

import jax
import jax.numpy as jnp
from jax.experimental import pallas as pl
from jax.experimental.pallas import tpu as pltpu

_EPS = 1e-5


def _bf(a):
    return a.astype(jnp.bfloat16)


def _bff(a):
    return a.astype(jnp.bfloat16).astype(jnp.float32)


def _stats_rows(n, c_out):
    cap = 2048 if c_out <= 256 else 512
    if n <= cap:
        return n
    for t in range(cap, 7, -8):
        if n % t == 0:
            return t
    return n


def _tile_rows(n, chunk, c_out):
    cap = 4096
    t = chunk
    while t * 2 <= cap and n % (t * 2) == 0:
        t *= 2
    return t


def _pass_kernel(*refs, stages, want_sq, want_extrema, want_gram,
                 gram_final_h, stats_rows):
    i = 0
    xr = refs[i][...]
    i += 1
    bb, tn = xr.shape[0], xr.shape[1]
    h = xr.reshape(bb * tn, xr.shape[2])
    for st in stages[:-1]:
        if st["batched"]:
            wb = refs[i][...]
            y = jnp.concatenate(
                [jnp.dot(h[m * tn:(m + 1) * tn], wb[m],
                         preferred_element_type=jnp.float32)
                 for m in range(bb)], axis=0)
        else:
            y = jnp.dot(h, refs[i][...], preferred_element_type=jnp.float32)
        i += 1
        if st["bias"]:
            blk = refs[i][...]
            y = jnp.concatenate(
                [y[m * tn:(m + 1) * tn] + blk[m] for m in range(bb)], axis=0)
            i += 1
        hb = y.astype(jnp.bfloat16)
        if st["affine"]:
            sc, sh = refs[i][...], refs[i + 1][...]
            i += 2
            h = _bf(jnp.maximum(hb.astype(jnp.float32) * sc + sh, 0.0))
        else:
            h = hb

    last = stages[-1]
    wl = refs[i][...]
    i += 1
    bl = None
    if last["bias"]:
        bl = refs[i][...]
        i += 1

    outs = list(refs[i:])
    step = pl.program_id(1)

    @pl.when(step == 0)
    def _():
        k = 0
        if want_sq:
            outs[k][...] = jnp.zeros_like(outs[k])
            outs[k + 1][...] = jnp.zeros_like(outs[k + 1])
            k += 2
        if want_gram:
            outs[k][...] = jnp.zeros_like(outs[k])
            outs[k + 1][...] = jnp.zeros_like(outs[k + 1])
            k += 2
        if want_extrema:
            outs[k][...] = jnp.full(outs[k].shape, -jnp.inf, jnp.float32)
            outs[k + 1][...] = jnp.full(outs[k + 1].shape, jnp.inf, jnp.float32)

    c_last = wl.shape[-1]
    csize = 512 if c_last >= 512 else c_last
    ones_row = jnp.ones((1, stats_rows), jnp.float32)
    hg = [None] * bb
    for c0 in range(0, c_last, csize):
        if last["batched"]:
            yc = [jnp.dot(h[m * tn:(m + 1) * tn], wl[m][:, c0:c0 + csize],
                          preferred_element_type=jnp.float32)
                  for m in range(bb)]
        else:
            ystk = jnp.dot(h, wl[:, c0:c0 + csize],
                           preferred_element_type=jnp.float32)
            yc = [ystk[m * tn:(m + 1) * tn] for m in range(bb)]
        for m in range(bb):
            y = yc[m]
            if bl is not None:
                y = y + bl[m][:, c0:c0 + csize]
            if gram_final_h:
                hg[m] = y.astype(jnp.bfloat16)
            k = 0
            if want_sq:
                for j in range(tn // stats_rows):
                    ys = y[j * stats_rows:(j + 1) * stats_rows]
                    outs[k][m, :, c0:c0 + csize] += jnp.dot(
                        ones_row, ys, preferred_element_type=jnp.float32)
                    outs[k + 1][m, :, c0:c0 + csize] += jnp.dot(
                        ones_row, ys * ys, preferred_element_type=jnp.float32)
                k += 2
            if want_gram:
                k += 2
            if want_extrema:
                outs[k][m, :, c0:c0 + csize] = jnp.maximum(
                    outs[k][m, :, c0:c0 + csize],
                    jnp.max(y, axis=0, keepdims=True))
                outs[k + 1][m, :, c0:c0 + csize] = jnp.minimum(
                    outs[k + 1][m, :, c0:c0 + csize],
                    jnp.min(y, axis=0, keepdims=True))

    if want_gram:
        k = 2 if want_sq else 0
        for m in range(bb):
            hgm = hg[m] if gram_final_h else h[m * tn:(m + 1) * tn]
            outs[k][m] += jax.lax.dot_general(
                hgm, hgm, (((0,), (0,)), ((), ())),
                preferred_element_type=jnp.float32)
            outs[k + 1][m] += jnp.sum(hgm.astype(jnp.float32), axis=0,
                                      keepdims=True)


def _run_pass(x, stage_params, want_sq=False, want_extrema=False,
              want_gram=False, gram_final_h=False):
    b, n, _ = x.shape
    c_last = stage_params[-1][0].shape[-1]
    chunk = _stats_rows(n, c_last)
    tn = _tile_rows(n, chunk, c_last)
    nt = n // tn
    bb = 2 if b % 2 == 0 else 1

    in_specs = [pl.BlockSpec((bb, tn, x.shape[2]), lambda bi, ni: (bi, ni, 0))]
    args = [x]
    stages = []
    for w, bias, aff in stage_params:
        c = w.shape[-1]
        if w.ndim == 3:
            in_specs.append(pl.BlockSpec((bb,) + w.shape[1:],
                                         lambda bi, ni: (bi, 0, 0)))
        else:
            in_specs.append(pl.BlockSpec(w.shape, lambda bi, ni: (0, 0)))
        args.append(_bf(w))
        if bias is not None:
            in_specs.append(pl.BlockSpec((bb, 1, c), lambda bi, ni: (bi, 0, 0)))
            args.append(bias.reshape(b, 1, c).astype(jnp.float32))
        if aff is not None:
            in_specs += [pl.BlockSpec((1, c), lambda bi, ni: (0, 0))] * 2
            args += [aff[0].reshape(1, c).astype(jnp.float32),
                     aff[1].reshape(1, c).astype(jnp.float32)]
        stages.append({"batched": w.ndim == 3, "bias": bias is not None,
                       "affine": aff is not None})

    out_shapes, out_specs = [], []

    def stat_out(c):
        out_shapes.append(jax.ShapeDtypeStruct((b, 1, c), jnp.float32))
        out_specs.append(pl.BlockSpec((bb, 1, c), lambda bi, ni: (bi, 0, 0)))

    if want_sq:
        stat_out(c_last)
        stat_out(c_last)
    if want_gram:
        kg = (stage_params[-1][0].shape[-1] if gram_final_h
              else stage_params[-1][0].shape[-2])
        out_shapes.append(jax.ShapeDtypeStruct((b, kg, kg), jnp.float32))
        out_specs.append(pl.BlockSpec((bb, kg, kg), lambda bi, ni: (bi, 0, 0)))
        stat_out(kg)
    if want_extrema:
        stat_out(c_last)
        stat_out(c_last)

    fn = functools.partial(_pass_kernel, stages=tuple(stages),
                           want_sq=want_sq, want_extrema=want_extrema,
                           want_gram=want_gram, gram_final_h=gram_final_h,
                           stats_rows=chunk)
    return pl.pallas_call(
        fn, out_shape=tuple(out_shapes), grid=(b // bb, nt),
        in_specs=in_specs, out_specs=tuple(out_specs),
        compiler_params=pltpu.CompilerParams(
            dimension_semantics=("parallel", "arbitrary")),
    )(*args)


def _head_kernel(x_ref, w1_ref, b1_ref, g1_ref, e1_ref, w2_ref, b2_ref,
                 g2_ref, e2_ref, w3_ref, b3_ref, o_ref, *, bn, softmax):
    h = x_ref[...].astype(jnp.float32)
    for w_ref, b_ref, g_ref, e_ref in ((w1_ref, b1_ref, g1_ref, e1_ref),
                                       (w2_ref, b2_ref, g2_ref, e2_ref)):
        y = jnp.dot(_bf(h), w_ref[...],
                    preferred_element_type=jnp.float32) + b_ref[...]
        if bn:
            mu = jnp.mean(y, axis=0, keepdims=True)
            v = jnp.mean(jnp.square(y - mu), axis=0, keepdims=True)
            y = (y - mu) * jax.lax.rsqrt(v + _EPS) * g_ref[...] + e_ref[...]
        h = jnp.maximum(y, 0.0)
    o = jnp.dot(_bf(h), w3_ref[...],
                preferred_element_type=jnp.float32) + b3_ref[...]
    if softmax:
        e = jnp.exp(o - jnp.max(o, axis=1, keepdims=True))
        o = e / jnp.sum(e, axis=1, keepdims=True)
    o_ref[...] = o


def _head(x, w1, b1, g1, e1, w2, b2, g2, e2, w3, b3, *, bn, softmax):
    b = x.shape[0]
    c1, c2, c3 = w1.shape[1], w2.shape[1], w3.shape[1]

    def v(a, c):
        return a.reshape(1, c).astype(jnp.float32)

    args = (x.astype(jnp.float32),
            _bf(w1), v(b1, c1), v(g1, c1), v(e1, c1),
            _bf(w2), v(b2, c2), v(g2, c2), v(e2, c2),
            _bf(w3), v(b3, c3))

    def full(shape):
        nd = len(shape)
        return pl.BlockSpec(shape, lambda i, _nd=nd: (0,) * _nd)

    return pl.pallas_call(
        functools.partial(_head_kernel, bn=bn, softmax=softmax),
        out_shape=jax.ShapeDtypeStruct((b, c3), jnp.float32),
        grid=(1,),
        in_specs=[full(a.shape) for a in args],
        out_specs=full((b, c3)),
        compiler_params=pltpu.CompilerParams(
            dimension_semantics=("arbitrary",)),
    )(*args)


def _stats_to_affine(s, q, cnt, gamma, beta):
    mean = jnp.sum(s, axis=0) / cnt
    var = jnp.maximum(jnp.sum(q, axis=0) / cnt - mean * mean, 0.0)
    sc = gamma * jax.lax.rsqrt(var + _EPS)
    return sc, beta - mean * sc


def _qf(g, w):
    return jnp.einsum("ij,ic,jc->c", g, w, w)


def kernel(x,
           pn_conv1_w, pn_conv1_b, pn_conv2_w, pn_conv2_b, pn_conv3_w, pn_conv3_b,
           pn_bn1_g, pn_bn1_b, pn_bn2_g, pn_bn2_b, pn_bn3_g, pn_bn3_b,
           cls_conv1_w, cls_conv1_b, cls_conv2_w, cls_conv2_b, cls_conv3_w, cls_conv3_b,
           t1_conv1_w, t1_conv1_b, t1_conv2_w, t1_conv2_b, t1_conv3_w, t1_conv3_b,
           t1_fc1_w, t1_fc1_b, t1_fc2_w, t1_fc2_b, t1_fc3_w, t1_fc3_b,
           t1_bn1_g, t1_bn1_b, t1_bn2_g, t1_bn2_b, t1_bn3_g, t1_bn3_b,
           t1_bn4_g, t1_bn4_b, t1_bn5_g, t1_bn5_b,
           t2_conv1_w, t2_conv1_b, t2_conv2_w, t2_conv2_b, t2_conv3_w, t2_conv3_b,
           t2_fc1_w, t2_fc1_b, t2_fc2_w, t2_fc2_b, t2_fc3_w, t2_fc3_b,
           t2_bn1_g, t2_bn1_b, t2_bn2_g, t2_bn2_b, t2_bn3_g, t2_bn3_b,
           t2_bn4_g, t2_bn4_b, t2_bn5_g, t2_bn5_b):
    b, n, pd = x.shape
    cnt = b * n

    xb = jnp.pad(_bf(x), ((0, 0), (0, 0), (0, 8 - pd)))

    w1t = jnp.pad(t1_conv1_w, ((0, 8 - pd), (0, 0)))
    s1, q1 = _run_pass(xb, [(w1t, None, None)], want_sq=True)
    sc1, sh1 = _stats_to_affine(s1[:, 0], q1[:, 0], cnt, t1_bn1_g, t1_bn1_b)

    s2, q2 = _run_pass(xb, [(w1t, None, (sc1, sh1)),
                            (t1_conv2_w, None, None)], want_sq=True)
    sc2, sh2 = _stats_to_affine(s2[:, 0], q2[:, 0], cnt, t1_bn2_g, t1_bn2_b)

    s3, q3, mx1, mn1 = _run_pass(
        xb, [(w1t, None, (sc1, sh1)), (t1_conv2_w, None, (sc2, sh2)),
             (t1_conv3_w, None, None)], want_sq=True, want_extrema=True)
    sc3, sh3 = _stats_to_affine(s3[:, 0], q3[:, 0], cnt, t1_bn3_g, t1_bn3_b)
    ext = jnp.where(sc3 >= 0.0, mx1[:, 0], mn1[:, 0])
    pooled = jnp.maximum(ext * sc3 + sh3, 0.0)

    t1m = (_head(pooled, t1_fc1_w, t1_fc1_b, t1_bn4_g, t1_bn4_b,
                 t1_fc2_w, t1_fc2_b, t1_bn5_g, t1_bn5_b,
                 t1_fc3_w, t1_fc3_b, bn=True, softmax=False)
           + jnp.eye(pd, dtype=jnp.float32).reshape(1, pd * pd)
           ).reshape(b, pd, pd)

    wf1 = jnp.pad(jnp.einsum("bij,jk->bik", t1m, pn_conv1_w),
                  ((0, 0), (0, 8 - pd), (0, 0)))
    s1m, q1m, gy1, csy1 = _run_pass(xb, [(wf1, None, None)],
                                    want_sq=True, want_gram=True,
                                    gram_final_h=True)
    sc1m, sh1m = _stats_to_affine(s1m[:, 0], q1m[:, 0], cnt,
                                  pn_bn1_g, pn_bn1_b)

    w1e = sc1m[:, None] * t2_conv1_w
    sa, qa = _run_pass(xb, [(wf1, None, None), (w1e, None, None)],
                       want_sq=True)
    sca, sha = _stats_to_affine(sa[:, 0], qa[:, 0], cnt, t2_bn1_g, t2_bn1_b)

    sb, qb = _run_pass(xb, [(wf1, None, None), (w1e, None, (sca, sha)),
                            (t2_conv2_w, None, None)], want_sq=True)
    scb, shb = _stats_to_affine(sb[:, 0], qb[:, 0], cnt, t2_bn2_g, t2_bn2_b)

    sc_, qc_, mx2, mn2 = _run_pass(
        xb, [(wf1, None, None), (w1e, None, (sca, sha)),
             (t2_conv2_w, None, (scb, shb)), (t2_conv3_w, None, None)],
        want_sq=True, want_extrema=True)
    scc, shc = _stats_to_affine(sc_[:, 0], qc_[:, 0], cnt, t2_bn3_g, t2_bn3_b)
    ext2 = jnp.where(scc >= 0.0, mx2[:, 0], mn2[:, 0])
    pooled2 = jnp.maximum(ext2 * scc + shc, 0.0)

    t2m = (_head(pooled2, t2_fc1_w, t2_fc1_b, t2_bn4_g, t2_bn4_b,
                 t2_fc2_w, t2_fc2_b, t2_bn5_g, t2_bn5_b,
                 t2_fc3_w, t2_fc3_b, bn=True, softmax=False)
           + jnp.eye(64, dtype=jnp.float32).reshape(1, 64 * 64)
           ).reshape(b, 64, 64)

    t2w2 = jnp.einsum("bij,jc->bic", t2m, pn_conv2_w)
    wf2 = sc1m[None, :, None] * t2w2
    bf2 = jnp.einsum("j,bjc->bc", sh1m, t2w2) + pn_conv2_b
    wf2f = _bff(wf2)
    lin = jnp.einsum("bi,bic->bc", csy1[:, 0], wf2f)
    s2m = jnp.sum(lin + n * bf2, axis=0)
    q2m = jnp.sum(jnp.einsum("bij,bic,bjc->bc", gy1, wf2f, wf2f)
                  + 2.0 * bf2 * lin + n * bf2 * bf2, axis=0)
    mean2 = s2m / cnt
    var2 = jnp.maximum(q2m / cnt - mean2 * mean2, 0.0)
    sc2m = pn_bn2_g * jax.lax.rsqrt(var2 + _EPS)
    wf3 = sc2m[:, None] * pn_conv3_w

    g2m, cs2m, mx3, mn3 = _run_pass(
        xb, [(wf1, None, None), (wf2, bf2, None), (wf3, None, None)],
        want_gram=True, want_extrema=True)
    wf3f = _bff(wf3)
    s3m = jnp.sum(cs2m[:, 0], axis=0) @ wf3f
    q3m = _qf(jnp.sum(g2m, axis=0), wf3f)
    mean3 = s3m / cnt
    var3 = jnp.maximum(q3m / cnt - mean3 * mean3, 0.0)
    sc3m = pn_bn3_g * jax.lax.rsqrt(var3 + _EPS)
    sh3m = pn_bn3_b - mean3 * sc3m
    pooled3 = jnp.where(sc3m >= 0.0, mx3[:, 0], mn3[:, 0]) * sc3m + sh3m

    ones = jnp.ones((cls_conv1_w.shape[1],), jnp.float32)
    zeros = jnp.zeros_like(ones)
    c2w = cls_conv2_w.shape[1]
    out = _head(pooled3, cls_conv1_w, cls_conv1_b, ones, zeros,
                cls_conv2_w, cls_conv2_b, ones[:c2w], zeros[:c2w],
                cls_conv3_w, cls_conv3_b, bn=False, softmax=True)
    return out.reshape(b, cls_conv3_w.shape[1], 1)


# speedup vs baseline: 1.8170x; 1.0176x over previous
import functools

import jax
import jax.numpy as jnp
from jax.experimental import pallas as pl
from jax.experimental.pallas import tpu as pltpu

_EPS = 1e-5


def _bf(a):
    return a.astype(jnp.bfloat16)


def _bff(a):
    return a.astype(jnp.bfloat16).astype(jnp.float32)


def _stats_rows(n, c_out):
    cap = 2048 if c_out <= 256 else 512
    if n <= cap:
        return n
    for t in range(cap, 7, -8):
        if n % t == 0:
            return t
    return n


def _tile_rows(n, chunk, c_out):
    cap = 4096
    t = chunk
    while t * 2 <= cap and n % (t * 2) == 0:
        t *= 2
    return t


def _pass_kernel(*refs, stages, want_sq, want_extrema, want_gram,
                 gram_final_h, stats_rows):
    i = 0
    xr = refs[i][...]
    i += 1
    bb, tn = xr.shape[0], xr.shape[1]
    h = xr.reshape(bb * tn, xr.shape[2])
    for st in stages[:-1]:
        if st["batched"]:
            wb = refs[i][...]
            y = jnp.concatenate(
                [jnp.dot(h[m * tn:(m + 1) * tn], wb[m],
                         preferred_element_type=jnp.float32)
                 for m in range(bb)], axis=0)
        else:
            y = jnp.dot(h, refs[i][...], preferred_element_type=jnp.float32)
        i += 1
        if st["bias"]:
            blk = refs[i][...]
            y = jnp.concatenate(
                [y[m * tn:(m + 1) * tn] + blk[m] for m in range(bb)], axis=0)
            i += 1
        hb = y.astype(jnp.bfloat16)
        if st["affine"]:
            sc, sh = refs[i][...], refs[i + 1][...]
            i += 2
            h = _bf(jnp.maximum(hb.astype(jnp.float32) * sc + sh, 0.0))
        else:
            h = hb

    last = stages[-1]
    wl = refs[i][...]
    i += 1
    bl = None
    if last["bias"]:
        bl = refs[i][...]
        i += 1

    outs = list(refs[i:])
    step = pl.program_id(1)

    @pl.when(step == 0)
    def _():
        k = 0
        if want_sq:
            outs[k][...] = jnp.zeros_like(outs[k])
            outs[k + 1][...] = jnp.zeros_like(outs[k + 1])
            k += 2
        if want_gram:
            outs[k][...] = jnp.zeros_like(outs[k])
            outs[k + 1][...] = jnp.zeros_like(outs[k + 1])
            k += 2
        if want_extrema:
            outs[k][...] = jnp.full(outs[k].shape, -jnp.inf, jnp.float32)
            outs[k + 1][...] = jnp.full(outs[k + 1].shape, jnp.inf, jnp.float32)

    c_last = wl.shape[-1]
    csize = 512 if c_last >= 512 else c_last
    ones_row = jnp.ones((1, stats_rows), jnp.float32)
    hg = [None] * bb
    for c0 in range(0, c_last, csize):
        if last["batched"]:
            yc = [jnp.dot(h[m * tn:(m + 1) * tn], wl[m][:, c0:c0 + csize],
                          preferred_element_type=jnp.float32)
                  for m in range(bb)]
        else:
            ystk = jnp.dot(h, wl[:, c0:c0 + csize],
                           preferred_element_type=jnp.float32)
            yc = [ystk[m * tn:(m + 1) * tn] for m in range(bb)]
        for m in range(bb):
            y = yc[m]
            if bl is not None:
                y = y + bl[m][:, c0:c0 + csize]
            if gram_final_h:
                hg[m] = y.astype(jnp.bfloat16)
            k = 0
            if want_sq:
                for j in range(tn // stats_rows):
                    ys = y[j * stats_rows:(j + 1) * stats_rows]
                    outs[k][m, :, c0:c0 + csize] += jnp.dot(
                        ones_row, ys, preferred_element_type=jnp.float32)
                    outs[k + 1][m, :, c0:c0 + csize] += jnp.dot(
                        ones_row, ys * ys, preferred_element_type=jnp.float32)
                k += 2
            if want_gram:
                k += 2
            if want_extrema:
                outs[k][m, :, c0:c0 + csize] = jnp.maximum(
                    outs[k][m, :, c0:c0 + csize],
                    jnp.max(y, axis=0, keepdims=True))
                outs[k + 1][m, :, c0:c0 + csize] = jnp.minimum(
                    outs[k + 1][m, :, c0:c0 + csize],
                    jnp.min(y, axis=0, keepdims=True))

    if want_gram:
        k = 2 if want_sq else 0
        for m in range(bb):
            hgm = hg[m] if gram_final_h else h[m * tn:(m + 1) * tn]
            outs[k][m] += jax.lax.dot_general(
                hgm, hgm, (((0,), (0,)), ((), ())),
                preferred_element_type=jnp.float32)
            outs[k + 1][m] += jnp.sum(hgm.astype(jnp.float32), axis=0,
                                      keepdims=True)


def _run_pass(x, stage_params, want_sq=False, want_extrema=False,
              want_gram=False, gram_final_h=False):
    b, n, _ = x.shape
    c_last = stage_params[-1][0].shape[-1]
    chunk = _stats_rows(n, c_last)
    tn = _tile_rows(n, chunk, c_last)
    nt = n // tn
    bb = 1
    for cand in (2,) if c_last >= 512 else (4, 2):
        if b % cand == 0:
            bb = cand
            break

    in_specs = [pl.BlockSpec((bb, tn, x.shape[2]), lambda bi, ni: (bi, ni, 0))]
    args = [x]
    stages = []
    for w, bias, aff in stage_params:
        c = w.shape[-1]
        if w.ndim == 3:
            in_specs.append(pl.BlockSpec((bb,) + w.shape[1:],
                                         lambda bi, ni: (bi, 0, 0)))
        else:
            in_specs.append(pl.BlockSpec(w.shape, lambda bi, ni: (0, 0)))
        args.append(_bf(w))
        if bias is not None:
            in_specs.append(pl.BlockSpec((bb, 1, c), lambda bi, ni: (bi, 0, 0)))
            args.append(bias.reshape(b, 1, c).astype(jnp.float32))
        if aff is not None:
            in_specs += [pl.BlockSpec((1, c), lambda bi, ni: (0, 0))] * 2
            args += [aff[0].reshape(1, c).astype(jnp.float32),
                     aff[1].reshape(1, c).astype(jnp.float32)]
        stages.append({"batched": w.ndim == 3, "bias": bias is not None,
                       "affine": aff is not None})

    out_shapes, out_specs = [], []

    def stat_out(c):
        out_shapes.append(jax.ShapeDtypeStruct((b, 1, c), jnp.float32))
        out_specs.append(pl.BlockSpec((bb, 1, c), lambda bi, ni: (bi, 0, 0)))

    if want_sq:
        stat_out(c_last)
        stat_out(c_last)
    if want_gram:
        kg = (stage_params[-1][0].shape[-1] if gram_final_h
              else stage_params[-1][0].shape[-2])
        out_shapes.append(jax.ShapeDtypeStruct((b, kg, kg), jnp.float32))
        out_specs.append(pl.BlockSpec((bb, kg, kg), lambda bi, ni: (bi, 0, 0)))
        stat_out(kg)
    if want_extrema:
        stat_out(c_last)
        stat_out(c_last)

    fn = functools.partial(_pass_kernel, stages=tuple(stages),
                           want_sq=want_sq, want_extrema=want_extrema,
                           want_gram=want_gram, gram_final_h=gram_final_h,
                           stats_rows=chunk)
    return pl.pallas_call(
        fn, out_shape=tuple(out_shapes), grid=(b // bb, nt),
        in_specs=in_specs, out_specs=tuple(out_specs),
        compiler_params=pltpu.CompilerParams(
            dimension_semantics=("parallel", "arbitrary")),
    )(*args)


def _head_kernel(x_ref, w1_ref, b1_ref, g1_ref, e1_ref, w2_ref, b2_ref,
                 g2_ref, e2_ref, w3_ref, b3_ref, o_ref, *, bn, softmax):
    h = x_ref[...].astype(jnp.float32)
    for w_ref, b_ref, g_ref, e_ref in ((w1_ref, b1_ref, g1_ref, e1_ref),
                                       (w2_ref, b2_ref, g2_ref, e2_ref)):
        y = jnp.dot(_bf(h), w_ref[...],
                    preferred_element_type=jnp.float32) + b_ref[...]
        if bn:
            mu = jnp.mean(y, axis=0, keepdims=True)
            v = jnp.mean(jnp.square(y - mu), axis=0, keepdims=True)
            y = (y - mu) * jax.lax.rsqrt(v + _EPS) * g_ref[...] + e_ref[...]
        h = jnp.maximum(y, 0.0)
    o = jnp.dot(_bf(h), w3_ref[...],
                preferred_element_type=jnp.float32) + b3_ref[...]
    if softmax:
        e = jnp.exp(o - jnp.max(o, axis=1, keepdims=True))
        o = e / jnp.sum(e, axis=1, keepdims=True)
    o_ref[...] = o


def _head(x, w1, b1, g1, e1, w2, b2, g2, e2, w3, b3, *, bn, softmax):
    b = x.shape[0]
    c1, c2, c3 = w1.shape[1], w2.shape[1], w3.shape[1]

    def v(a, c):
        return a.reshape(1, c).astype(jnp.float32)

    args = (x.astype(jnp.float32),
            _bf(w1), v(b1, c1), v(g1, c1), v(e1, c1),
            _bf(w2), v(b2, c2), v(g2, c2), v(e2, c2),
            _bf(w3), v(b3, c3))

    def full(shape):
        nd = len(shape)
        return pl.BlockSpec(shape, lambda i, _nd=nd: (0,) * _nd)

    return pl.pallas_call(
        functools.partial(_head_kernel, bn=bn, softmax=softmax),
        out_shape=jax.ShapeDtypeStruct((b, c3), jnp.float32),
        grid=(1,),
        in_specs=[full(a.shape) for a in args],
        out_specs=full((b, c3)),
        compiler_params=pltpu.CompilerParams(
            dimension_semantics=("arbitrary",)),
    )(*args)


def _stats_to_affine(s, q, cnt, gamma, beta):
    mean = jnp.sum(s, axis=0) / cnt
    var = jnp.maximum(jnp.sum(q, axis=0) / cnt - mean * mean, 0.0)
    sc = gamma * jax.lax.rsqrt(var + _EPS)
    return sc, beta - mean * sc


def _qf(g, w):
    return jnp.einsum("ij,ic,jc->c", g, w, w)


def kernel(x,
           pn_conv1_w, pn_conv1_b, pn_conv2_w, pn_conv2_b, pn_conv3_w, pn_conv3_b,
           pn_bn1_g, pn_bn1_b, pn_bn2_g, pn_bn2_b, pn_bn3_g, pn_bn3_b,
           cls_conv1_w, cls_conv1_b, cls_conv2_w, cls_conv2_b, cls_conv3_w, cls_conv3_b,
           t1_conv1_w, t1_conv1_b, t1_conv2_w, t1_conv2_b, t1_conv3_w, t1_conv3_b,
           t1_fc1_w, t1_fc1_b, t1_fc2_w, t1_fc2_b, t1_fc3_w, t1_fc3_b,
           t1_bn1_g, t1_bn1_b, t1_bn2_g, t1_bn2_b, t1_bn3_g, t1_bn3_b,
           t1_bn4_g, t1_bn4_b, t1_bn5_g, t1_bn5_b,
           t2_conv1_w, t2_conv1_b, t2_conv2_w, t2_conv2_b, t2_conv3_w, t2_conv3_b,
           t2_fc1_w, t2_fc1_b, t2_fc2_w, t2_fc2_b, t2_fc3_w, t2_fc3_b,
           t2_bn1_g, t2_bn1_b, t2_bn2_g, t2_bn2_b, t2_bn3_g, t2_bn3_b,
           t2_bn4_g, t2_bn4_b, t2_bn5_g, t2_bn5_b):
    b, n, pd = x.shape
    cnt = b * n

    xb = jnp.pad(_bf(x), ((0, 0), (0, 0), (0, 8 - pd)))

    w1t = jnp.pad(t1_conv1_w, ((0, 8 - pd), (0, 0)))
    s1, q1 = _run_pass(xb, [(w1t, None, None)], want_sq=True)
    sc1, sh1 = _stats_to_affine(s1[:, 0], q1[:, 0], cnt, t1_bn1_g, t1_bn1_b)

    s2, q2 = _run_pass(xb, [(w1t, None, (sc1, sh1)),
                            (t1_conv2_w, None, None)], want_sq=True)
    sc2, sh2 = _stats_to_affine(s2[:, 0], q2[:, 0], cnt, t1_bn2_g, t1_bn2_b)

    s3, q3, mx1, mn1 = _run_pass(
        xb, [(w1t, None, (sc1, sh1)), (t1_conv2_w, None, (sc2, sh2)),
             (t1_conv3_w, None, None)], want_sq=True, want_extrema=True)
    sc3, sh3 = _stats_to_affine(s3[:, 0], q3[:, 0], cnt, t1_bn3_g, t1_bn3_b)
    ext = jnp.where(sc3 >= 0.0, mx1[:, 0], mn1[:, 0])
    pooled = jnp.maximum(ext * sc3 + sh3, 0.0)

    t1m = (_head(pooled, t1_fc1_w, t1_fc1_b, t1_bn4_g, t1_bn4_b,
                 t1_fc2_w, t1_fc2_b, t1_bn5_g, t1_bn5_b,
                 t1_fc3_w, t1_fc3_b, bn=True, softmax=False)
           + jnp.eye(pd, dtype=jnp.float32).reshape(1, pd * pd)
           ).reshape(b, pd, pd)

    wf1 = jnp.pad(jnp.einsum("bij,jk->bik", t1m, pn_conv1_w),
                  ((0, 0), (0, 8 - pd), (0, 0)))
    s1m, q1m, gy1, csy1 = _run_pass(xb, [(wf1, None, None)],
                                    want_sq=True, want_gram=True,
                                    gram_final_h=True)
    sc1m, sh1m = _stats_to_affine(s1m[:, 0], q1m[:, 0], cnt,
                                  pn_bn1_g, pn_bn1_b)

    w1e = sc1m[:, None] * t2_conv1_w
    sa, qa = _run_pass(xb, [(wf1, None, None), (w1e, None, None)],
                       want_sq=True)
    sca, sha = _stats_to_affine(sa[:, 0], qa[:, 0], cnt, t2_bn1_g, t2_bn1_b)

    sb, qb = _run_pass(xb, [(wf1, None, None), (w1e, None, (sca, sha)),
                            (t2_conv2_w, None, None)], want_sq=True)
    scb, shb = _stats_to_affine(sb[:, 0], qb[:, 0], cnt, t2_bn2_g, t2_bn2_b)

    sc_, qc_, mx2, mn2 = _run_pass(
        xb, [(wf1, None, None), (w1e, None, (sca, sha)),
             (t2_conv2_w, None, (scb, shb)), (t2_conv3_w, None, None)],
        want_sq=True, want_extrema=True)
    scc, shc = _stats_to_affine(sc_[:, 0], qc_[:, 0], cnt, t2_bn3_g, t2_bn3_b)
    ext2 = jnp.where(scc >= 0.0, mx2[:, 0], mn2[:, 0])
    pooled2 = jnp.maximum(ext2 * scc + shc, 0.0)

    t2m = (_head(pooled2, t2_fc1_w, t2_fc1_b, t2_bn4_g, t2_bn4_b,
                 t2_fc2_w, t2_fc2_b, t2_bn5_g, t2_bn5_b,
                 t2_fc3_w, t2_fc3_b, bn=True, softmax=False)
           + jnp.eye(64, dtype=jnp.float32).reshape(1, 64 * 64)
           ).reshape(b, 64, 64)

    t2w2 = jnp.einsum("bij,jc->bic", t2m, pn_conv2_w)
    wf2 = sc1m[None, :, None] * t2w2
    bf2 = jnp.einsum("j,bjc->bc", sh1m, t2w2) + pn_conv2_b
    wf2f = _bff(wf2)
    lin = jnp.einsum("bi,bic->bc", csy1[:, 0], wf2f)
    s2m = jnp.sum(lin + n * bf2, axis=0)
    q2m = jnp.sum(jnp.einsum("bij,bic,bjc->bc", gy1, wf2f, wf2f)
                  + 2.0 * bf2 * lin + n * bf2 * bf2, axis=0)
    mean2 = s2m / cnt
    var2 = jnp.maximum(q2m / cnt - mean2 * mean2, 0.0)
    sc2m = pn_bn2_g * jax.lax.rsqrt(var2 + _EPS)
    wf3 = sc2m[:, None] * pn_conv3_w

    g2m, cs2m, mx3, mn3 = _run_pass(
        xb, [(wf1, None, None), (wf2, bf2, None), (wf3, None, None)],
        want_gram=True, want_extrema=True)
    wf3f = _bff(wf3)
    s3m = jnp.sum(cs2m[:, 0], axis=0) @ wf3f
    q3m = _qf(jnp.sum(g2m, axis=0), wf3f)
    mean3 = s3m / cnt
    var3 = jnp.maximum(q3m / cnt - mean3 * mean3, 0.0)
    sc3m = pn_bn3_g * jax.lax.rsqrt(var3 + _EPS)
    sh3m = pn_bn3_b - mean3 * sc3m
    pooled3 = jnp.where(sc3m >= 0.0, mx3[:, 0], mn3[:, 0]) * sc3m + sh3m

    ones = jnp.ones((cls_conv1_w.shape[1],), jnp.float32)
    zeros = jnp.zeros_like(ones)
    c2w = cls_conv2_w.shape[1]
    out = _head(pooled3, cls_conv1_w, cls_conv1_b, ones, zeros,
                cls_conv2_w, cls_conv2_b, ones[:c2w], zeros[:c2w],
                cls_conv3_w, cls_conv3_b, bn=False, softmax=True)
    return out.reshape(b, cls_conv3_w.shape[1], 1)
